# Initial kernel scaffold; baseline (speedup 1.0000x reference)
import jax
import jax.numpy as jnp
from jax import lax
from jax.experimental import pallas as pl
from jax.experimental.pallas import tpu as pltpu
from jax.experimental.pallas import tpu_sc as plsc

f32 = jnp.float32

N = 10000
D = 128
E = 320000

NC = 2
NS = 16
NW = NC * NS

G = 128
GPW = 79
EPAD = NW * GPW * G
R2D = EPAD // G

NPAD = 10016
ZROWS = NPAD // NW
TROWS = NPAD // NS

TBLK = 2504

_mesh = plsc.VectorSubcoreMesh(core_axis_name="c", subcore_axis_name="s")


def _worker_ids():
    cid = lax.axis_index("c")
    sid = lax.axis_index("s")
    return cid, sid, cid * NS + sid


def _deg_body(dst_hbm, out_hbm, idx_v, ones_v, zbuf, acc):
    cid, sid, w = _worker_ids()
    one = jnp.ones((16,), f32)
    zero = jnp.zeros((16,), f32)

    def fill_ones(i, _):
        ones_v[i] = one
        return 0

    lax.fori_loop(0, G, fill_ones, 0)

    def fill_z(i, _):
        zbuf[i] = zero
        return 0

    lax.fori_loop(0, ZROWS, fill_z, 0)
    pltpu.sync_copy(zbuf, acc.at[pl.ds(sid * TROWS, ZROWS)])
    pltpu.sync_copy(zbuf, acc.at[pl.ds(sid * TROWS + ZROWS, ZROWS)])
    pltpu.sync_copy(dst_hbm.at[pl.ds(w * GPW, GPW)], idx_v)
    plsc.subcore_barrier()

    def body(j, _):
        pltpu.sync_copy(ones_v, acc.at[idx_v.at[j]], add=True)
        return 0

    lax.fori_loop(0, GPW, body, 0)
    plsc.subcore_barrier()
    pltpu.sync_copy(acc.at[pl.ds(sid * TROWS, TROWS)],
                    out_hbm.at[cid, pl.ds(sid * TROWS, TROWS)])


_deg_call = pl.kernel(
    _deg_body,
    out_type=jax.ShapeDtypeStruct((NC, NPAD, 16), f32),
    mesh=_mesh,
    scratch_types=[
        pltpu.VMEM((GPW, G), jnp.int32),
        pltpu.VMEM((G, 16), f32),
        pltpu.VMEM((ZROWS, 16), f32),
        pltpu.VMEM_SHARED((NPAD, 16), f32),
    ],
)


def _scatter_body(xs_hbm, src_hbm, dst_hbm, out_hbm,
                  sidx, didx, rows, zbuf, acc, sem):
    cid, sid, w = _worker_ids()
    zero = jnp.zeros((16,), f32)

    def fill_z(i, _):
        for k in range(D // 16):
            zbuf[i, pl.ds(k * 16, 16)] = zero
        return 0

    lax.fori_loop(0, ZROWS, fill_z, 0)
    pltpu.sync_copy(zbuf, acc.at[pl.ds(sid * TROWS, ZROWS)])
    pltpu.sync_copy(zbuf, acc.at[pl.ds(sid * TROWS + ZROWS, ZROWS)])
    pltpu.sync_copy(src_hbm.at[pl.ds(w * GPW, GPW)], sidx)
    pltpu.sync_copy(dst_hbm.at[pl.ds(w * GPW, GPW)], didx)
    plsc.subcore_barrier()

    def body(j, _):
        pltpu.async_copy(xs_hbm.at[sidx.at[j]], rows, sem).wait()
        pltpu.sync_copy(rows, acc.at[didx.at[j]], add=True)
        return 0

    lax.fori_loop(0, GPW, body, 0)
    plsc.subcore_barrier()
    pltpu.sync_copy(acc.at[pl.ds(sid * TROWS, TROWS)],
                    out_hbm.at[cid, pl.ds(sid * TROWS, TROWS)])


_scatter_call = pl.kernel(
    _scatter_body,
    out_type=jax.ShapeDtypeStruct((NC, NPAD, D), f32),
    mesh=_mesh,
    scratch_types=[
        pltpu.VMEM((GPW, G), jnp.int32),
        pltpu.VMEM((GPW, G), jnp.int32),
        pltpu.VMEM((G, D), f32),
        pltpu.VMEM((ZROWS, D), f32),
        pltpu.VMEM_SHARED((NPAD, D), f32),
        pltpu.SemaphoreType.DMA,
    ],
)


def _score_body(h_hbm, src_hbm, dst_hbm, out_hbm,
                sidx, didx, rows, cols, sv, sem1, sem2):
    cid, sid, w = _worker_ids()
    pltpu.sync_copy(src_hbm.at[pl.ds(w * GPW, GPW)], sidx)
    pltpu.sync_copy(dst_hbm.at[pl.ds(w * GPW, GPW)], didx)

    def group(j, _):
        c1 = pltpu.async_copy(h_hbm.at[sidx.at[j]], rows, sem1)
        c2 = pltpu.async_copy(h_hbm.at[didx.at[j]], cols, sem2)
        c1.wait()
        c2.wait()

        def edot(e, _):
            acc = rows[e, pl.ds(0, 16)] * cols[e, pl.ds(0, 16)]
            for k in range(1, D // 16):
                acc = acc + rows[e, pl.ds(k * 16, 16)] * cols[e, pl.ds(k * 16, 16)]
            sv[e] = jnp.sum(acc)
            return 0

        lax.fori_loop(0, G, edot, 0)
        for t in range(G // 16):
            v = sv[pl.ds(t * 16, 16)]
            sv[pl.ds(t * 16, 16)] = 1.0 / (1.0 + jnp.exp(-v))
        pltpu.sync_copy(sv, out_hbm.at[w * GPW + j])
        return 0

    lax.fori_loop(0, GPW, group, 0)


_score_call = pl.kernel(
    _score_body,
    out_type=jax.ShapeDtypeStruct((R2D, G), f32),
    mesh=_mesh,
    scratch_types=[
        pltpu.VMEM((GPW, G), jnp.int32),
        pltpu.VMEM((GPW, G), jnp.int32),
        pltpu.VMEM((G, D), f32),
        pltpu.VMEM((G, D), f32),
        pltpu.VMEM((G,), f32),
        pltpu.SemaphoreType.DMA,
        pltpu.SemaphoreType.DMA,
    ],
)


def _dinv_from(da_ref, db_ref):
    deg = da_ref[0, :, 0:1] + db_ref[0, :, 0:1] + 1.0
    return lax.rsqrt(deg)


def _tc1_body(x_ref, w_ref, da_ref, db_ref, o_ref):
    dinv = _dinv_from(da_ref, db_ref)
    o_ref[...] = jnp.dot(x_ref[...], w_ref[...],
                         preferred_element_type=f32) * dinv


def _tc2_body(a0_ref, a1_ref, xs_ref, b_ref, w_ref, da_ref, db_ref, o_ref):
    dinv = _dinv_from(da_ref, db_ref)
    h = dinv * (a0_ref[0] + a1_ref[0] + xs_ref[...]) + b_ref[...]
    h = jnp.maximum(h, 0.0)
    o_ref[...] = jnp.dot(h, w_ref[...], preferred_element_type=f32) * dinv


def _tc3_body(a0_ref, a1_ref, xs_ref, b_ref, da_ref, db_ref, o_ref):
    dinv = _dinv_from(da_ref, db_ref)
    o_ref[...] = dinv * (a0_ref[0] + a1_ref[0] + xs_ref[...]) + b_ref[...]


_blk = pl.BlockSpec((TBLK, D), lambda i: (i, 0))
_blkW = pl.BlockSpec((D, D), lambda i: (0, 0))
_blkb = pl.BlockSpec((1, D), lambda i: (0, 0))
_blk_dA = pl.BlockSpec((1, TBLK, 16), lambda i: (0, i, 0))
_blk_dB = pl.BlockSpec((1, TBLK, 16), lambda i: (1, i, 0))
_blk_aA = pl.BlockSpec((1, TBLK, D), lambda i: (0, i, 0))
_blk_aB = pl.BlockSpec((1, TBLK, D), lambda i: (1, i, 0))
_out_nd = jax.ShapeDtypeStruct((NPAD, D), f32)
_grid = (NPAD // TBLK,)


def _tc1(x_pad, W1, deg):
    return pl.pallas_call(
        _tc1_body, grid=_grid,
        in_specs=[_blk, _blkW, _blk_dA, _blk_dB],
        out_specs=_blk, out_shape=_out_nd,
    )(x_pad, W1, deg, deg)


def _tc2(acc1, xs1, b1, W2, deg):
    return pl.pallas_call(
        _tc2_body, grid=_grid,
        in_specs=[_blk_aA, _blk_aB, _blk, _blkb, _blkW, _blk_dA, _blk_dB],
        out_specs=_blk, out_shape=_out_nd,
    )(acc1, acc1, xs1, b1, W2, deg, deg)


def _tc3(acc2, xs2, b2, deg):
    return pl.pallas_call(
        _tc3_body, grid=_grid,
        in_specs=[_blk_aA, _blk_aB, _blk, _blkb, _blk_dA, _blk_dB],
        out_specs=_blk, out_shape=_out_nd,
    )(acc2, acc2, xs2, b2, deg, deg)


def kernel(x, edge_index, W1, b1, W2, b2):
    src = edge_index[0].astype(jnp.int32)
    dst = edge_index[1].astype(jnp.int32)
    pad = jnp.full((EPAD - E,), N, jnp.int32)
    src2d = jnp.concatenate([src, pad]).reshape(R2D, G)
    dst2d = jnp.concatenate([dst, pad]).reshape(R2D, G)
    x_pad = jnp.concatenate([x, jnp.zeros((NPAD - N, D), f32)], axis=0)

    deg = _deg_call(dst2d)
    xs1 = _tc1(x_pad, W1, deg)
    acc1 = _scatter_call(xs1, src2d, dst2d)
    xs2 = _tc2(acc1, xs1, b1.reshape(1, D), W2, deg)
    acc2 = _scatter_call(xs2, src2d, dst2d)
    h2 = _tc3(acc2, xs2, b2.reshape(1, D), deg)
    scores = _score_call(h2, src2d, dst2d)
    return scores.reshape(-1)[:E]


# speedup vs baseline: 5.2862x; 5.2862x over previous
import functools

import jax
import jax.numpy as jnp
from jax import lax
from jax.experimental import pallas as pl
from jax.experimental.pallas import tpu as pltpu
from jax.experimental.pallas import tpu_sc as plsc

f32 = jnp.float32

N = 10000
D = 128
E = 320000

NC = 2
NS = 16
NW = NC * NS

G = 128
GPW = 80
EPAD = NW * GPW * G
R2D = EPAD // G

NPAD = 10240
ZROWS = NPAD // NW
TROWS = NPAD // NS

TBLK = 2560

CG = 8
NCHUNK = GPW // CG

@functools.cache
def _mesh():
    return plsc.VectorSubcoreMesh(core_axis_name="c", subcore_axis_name="s",
                                  num_cores=NC, num_subcores=NS)


def _worker_ids():
    cid = lax.axis_index("c")
    sid = lax.axis_index("s")
    return cid, sid, cid * NS + sid


def _deg_body(dst_hbm, out_hbm, idx_v, ones_v, acc):
    cid, sid, w = _worker_ids()
    one = jnp.ones((16,), f32)
    zero = jnp.zeros((16,), f32)

    def fill(val):
        def go(i, _):
            ones_v[i] = val
            return 0
        lax.fori_loop(0, G, go, 0)

    fill(zero)
    for t in range(TROWS // G):
        pltpu.sync_copy(ones_v, acc.at[pl.ds(sid * TROWS + t * G, G)])
    fill(one)
    pltpu.sync_copy(dst_hbm.at[pl.ds(w * GPW, GPW)], idx_v)
    plsc.subcore_barrier()

    def body(j, _):
        pltpu.sync_copy(ones_v, acc.at[idx_v.at[j]], add=True)
        return 0

    lax.fori_loop(0, GPW, body, 0)
    plsc.subcore_barrier()
    pltpu.sync_copy(acc.at[pl.ds(sid * TROWS, TROWS)],
                    out_hbm.at[cid, pl.ds(sid * TROWS, TROWS)])


@functools.cache
def _deg_call():
    return pl.kernel(
        _deg_body,
        out_type=jax.ShapeDtypeStruct((NC, NPAD, 16), f32),
        mesh=_mesh(),
        compiler_params=pltpu.CompilerParams(needs_layout_passes=False),
        scratch_types=[
            pltpu.VMEM((GPW, G), jnp.int32),
            pltpu.VMEM((G, 16), f32),
            pltpu.VMEM_SHARED((NPAD, 16), f32),
        ],
    )


def _scatter_body(xs_hbm, src_hbm, dst_hbm, out_hbm,
                  sidx, didx, rows, acc, sem):
    cid, sid, w = _worker_ids()
    zero = jnp.zeros((16,), f32)

    def fill_z(i, _):
        for k in range(D // 16):
            rows[i, pl.ds(k * 16, 16)] = zero
        return 0

    lax.fori_loop(0, G, fill_z, 0)
    for t in range(TROWS // G):
        pltpu.sync_copy(rows, acc.at[pl.ds(sid * TROWS + t * G, G)])
    plsc.subcore_barrier()

    def chunk(c, _):
        pltpu.sync_copy(src_hbm.at[pl.ds(w * GPW + c * CG, CG)], sidx)
        pltpu.sync_copy(dst_hbm.at[pl.ds(w * GPW + c * CG, CG)], didx)

        def body(j, _):
            pltpu.async_copy(xs_hbm.at[sidx.at[j]], rows, sem).wait()
            pltpu.sync_copy(rows, acc.at[didx.at[j]], add=True)
            return 0

        lax.fori_loop(0, CG, body, 0)
        return 0

    lax.fori_loop(0, NCHUNK, chunk, 0)
    plsc.subcore_barrier()
    pltpu.sync_copy(acc.at[pl.ds(sid * TROWS, TROWS)],
                    out_hbm.at[cid, pl.ds(sid * TROWS, TROWS)])


@functools.cache
def _scatter_call():
    return pl.kernel(
        _scatter_body,
        out_type=jax.ShapeDtypeStruct((NC, NPAD, D), f32),
        mesh=_mesh(),
        compiler_params=pltpu.CompilerParams(needs_layout_passes=False),
        scratch_types=[
            pltpu.VMEM((CG, G), jnp.int32),
            pltpu.VMEM((CG, G), jnp.int32),
            pltpu.VMEM((G, D), f32),
            pltpu.VMEM_SHARED((NPAD, D), f32),
            pltpu.SemaphoreType.DMA,
        ],
    )


def _score_body(h_hbm, src_hbm, dst_hbm, out_hbm,
                sidx, didx, rows, cols, sv, sem1, sem2):
    cid, sid, w = _worker_ids()
    pltpu.sync_copy(src_hbm.at[pl.ds(w * GPW, GPW)], sidx)
    pltpu.sync_copy(dst_hbm.at[pl.ds(w * GPW, GPW)], didx)

    def group(j, _):
        c1 = pltpu.async_copy(h_hbm.at[sidx.at[j]], rows, sem1)
        c2 = pltpu.async_copy(h_hbm.at[didx.at[j]], cols, sem2)
        c1.wait()
        c2.wait()

        lanes = lax.broadcasted_iota(jnp.int32, (16,), 0)

        def sub(s, _):
            def edot(t, vec):
                e = s * 16 + t
                acc = rows[e, pl.ds(0, 16)] * cols[e, pl.ds(0, 16)]
                for k in range(1, D // 16):
                    acc = acc + rows[e, pl.ds(k * 16, 16)] * cols[e, pl.ds(k * 16, 16)]
                sc = jnp.sum(acc)
                return jnp.where(lanes == jnp.full((16,), t, jnp.int32), sc, vec)

            vec = lax.fori_loop(0, 16, edot, jnp.zeros((16,), f32))
            sv[pl.ds(s * 16, 16)] = 1.0 / (1.0 + jnp.exp(-vec))
            return 0

        lax.fori_loop(0, G // 16, sub, 0)
        pltpu.sync_copy(sv, out_hbm.at[pl.ds((w * GPW + j) * G, G)])
        return 0

    lax.fori_loop(0, GPW, group, 0)


@functools.cache
def _score_call():
    return pl.kernel(
        _score_body,
        out_type=jax.ShapeDtypeStruct((EPAD,), f32),
        mesh=_mesh(),
        compiler_params=pltpu.CompilerParams(needs_layout_passes=False),
        scratch_types=[
            pltpu.VMEM((GPW, G), jnp.int32),
            pltpu.VMEM((GPW, G), jnp.int32),
            pltpu.VMEM((G, D), f32),
            pltpu.VMEM((G, D), f32),
            pltpu.VMEM((G,), f32),
            pltpu.SemaphoreType.DMA,
            pltpu.SemaphoreType.DMA,
        ],
    )


def _dinv_from(da_ref, db_ref):
    deg = da_ref[0, :, 0:1] + db_ref[0, :, 0:1] + 1.0
    return lax.rsqrt(deg)


def _tc1_body(x_ref, w_ref, da_ref, db_ref, o_ref):
    dinv = _dinv_from(da_ref, db_ref)
    o_ref[...] = jnp.dot(x_ref[...], w_ref[...],
                         preferred_element_type=f32) * dinv


def _tc2_body(a0_ref, a1_ref, xs_ref, b_ref, w_ref, da_ref, db_ref, o_ref):
    dinv = _dinv_from(da_ref, db_ref)
    h = dinv * (a0_ref[0] + a1_ref[0] + xs_ref[...]) + b_ref[...]
    h = jnp.maximum(h, 0.0)
    o_ref[...] = jnp.dot(h, w_ref[...], preferred_element_type=f32) * dinv


def _tc3_body(a0_ref, a1_ref, xs_ref, b_ref, da_ref, db_ref, o_ref):
    dinv = _dinv_from(da_ref, db_ref)
    o_ref[...] = dinv * (a0_ref[0] + a1_ref[0] + xs_ref[...]) + b_ref[...]


_blk = pl.BlockSpec((TBLK, D), lambda i: (i, 0))
_blkW = pl.BlockSpec((D, D), lambda i: (0, 0))
_blkb = pl.BlockSpec((1, D), lambda i: (0, 0))
_blk_dA = pl.BlockSpec((1, TBLK, 16), lambda i: (0, i, 0))
_blk_dB = pl.BlockSpec((1, TBLK, 16), lambda i: (1, i, 0))
_blk_aA = pl.BlockSpec((1, TBLK, D), lambda i: (0, i, 0))
_blk_aB = pl.BlockSpec((1, TBLK, D), lambda i: (1, i, 0))
_out_nd = jax.ShapeDtypeStruct((NPAD, D), f32)
_grid = (NPAD // TBLK,)


def _tc1(x_pad, W1, deg):
    return pl.pallas_call(
        _tc1_body, grid=_grid,
        in_specs=[_blk, _blkW, _blk_dA, _blk_dB],
        out_specs=_blk, out_shape=_out_nd,
    )(x_pad, W1, deg, deg)


def _tc2(acc1, xs1, b1, W2, deg):
    return pl.pallas_call(
        _tc2_body, grid=_grid,
        in_specs=[_blk_aA, _blk_aB, _blk, _blkb, _blkW, _blk_dA, _blk_dB],
        out_specs=_blk, out_shape=_out_nd,
    )(acc1, acc1, xs1, b1, W2, deg, deg)


def _tc3(acc2, xs2, b2, deg):
    return pl.pallas_call(
        _tc3_body, grid=_grid,
        in_specs=[_blk_aA, _blk_aB, _blk, _blkb, _blk_dA, _blk_dB],
        out_specs=_blk, out_shape=_out_nd,
    )(acc2, acc2, xs2, b2, deg, deg)


def kernel(x, edge_index, W1, b1, W2, b2):
    src = edge_index[0].astype(jnp.int32)
    dst = edge_index[1].astype(jnp.int32)
    pad = jnp.full((EPAD - E,), N, jnp.int32)
    src2d = jnp.concatenate([src, pad]).reshape(R2D, G)
    dst2d = jnp.concatenate([dst, pad]).reshape(R2D, G)
    x_pad = jnp.concatenate([x, jnp.zeros((NPAD - N, D), f32)], axis=0)

    deg = _deg_call()(dst2d)
    xs1 = _tc1(x_pad, W1, deg)
    acc1 = _scatter_call()(xs1, src2d, dst2d)
    xs2 = _tc2(acc1, xs1, b1.reshape(1, D), W2, deg)
    acc2 = _scatter_call()(xs2, src2d, dst2d)
    h2 = _tc3(acc2, xs2, b2.reshape(1, D), deg)
    scores = _score_call()(h2, src2d, dst2d)
    return scores[:E]


# speedup vs baseline: 5.5738x; 1.0544x over previous
import functools

import jax
import jax.numpy as jnp
from jax import lax
from jax.experimental import pallas as pl
from jax.experimental.pallas import tpu as pltpu
from jax.experimental.pallas import tpu_sc as plsc

f32 = jnp.float32

N = 10000
D = 128
E = 320000

NC = 2
NS = 16
NW = NC * NS

G = 128
GPW = 80
EPAD = NW * GPW * G
R2D = EPAD // G

NPAD = 10240
ZROWS = NPAD // NW
TROWS = NPAD // NS

TBLK = 2560

CG = 8
NCHUNK = GPW // CG

@functools.cache
def _mesh():
    return plsc.VectorSubcoreMesh(core_axis_name="c", subcore_axis_name="s",
                                  num_cores=NC, num_subcores=NS)


def _worker_ids():
    cid = lax.axis_index("c")
    sid = lax.axis_index("s")
    return cid, sid, cid * NS + sid


def _deg_body(dst_hbm, out_hbm, idx_v, ones_v, acc):
    cid, sid, w = _worker_ids()
    one = jnp.ones((16,), f32)
    zero = jnp.zeros((16,), f32)

    def fill(val):
        def go(i, _):
            ones_v[i] = val
            return 0
        lax.fori_loop(0, G, go, 0)

    fill(zero)
    for t in range(TROWS // G):
        pltpu.sync_copy(ones_v, acc.at[pl.ds(sid * TROWS + t * G, G)])
    fill(one)
    pltpu.sync_copy(dst_hbm.at[pl.ds(w * GPW, GPW)], idx_v)
    plsc.subcore_barrier()

    def body(j, _):
        pltpu.sync_copy(ones_v, acc.at[idx_v.at[j]], add=True)
        return 0

    lax.fori_loop(0, GPW, body, 0)
    plsc.subcore_barrier()
    pltpu.sync_copy(acc.at[pl.ds(sid * TROWS, TROWS)],
                    out_hbm.at[cid, pl.ds(sid * TROWS, TROWS)])


@functools.cache
def _deg_call():
    return pl.kernel(
        _deg_body,
        out_type=jax.ShapeDtypeStruct((NC, NPAD, 16), f32),
        mesh=_mesh(),
        compiler_params=pltpu.CompilerParams(needs_layout_passes=False),
        scratch_types=[
            pltpu.VMEM((GPW, G), jnp.int32),
            pltpu.VMEM((G, 16), f32),
            pltpu.VMEM_SHARED((NPAD, 16), f32),
        ],
    )


def _scatter_body(xs_hbm, src_hbm, dst_hbm, out_hbm,
                  sem, sidx, didx, rowsbuf, acc):
    cid, sid, w = _worker_ids()
    zero = jnp.zeros((16,), f32)
    sems = (sem.at[0], sem.at[1])

    def fill_z(i, _):
        for k in range(D // 16):
            rowsbuf[0, i, pl.ds(k * 16, 16)] = zero
        return 0

    lax.fori_loop(0, G, fill_z, 0)
    for t in range(TROWS // G):
        pltpu.sync_copy(rowsbuf.at[0], acc.at[pl.ds(sid * TROWS + t * G, G)])
    plsc.subcore_barrier()

    def chunk(c, _):
        pltpu.sync_copy(src_hbm.at[pl.ds(w * GPW + c * CG, CG)], sidx)
        pltpu.sync_copy(dst_hbm.at[pl.ds(w * GPW + c * CG, CG)], didx)
        cp = [pltpu.async_copy(xs_hbm.at[sidx.at[0]], rowsbuf.at[0], sems[0]),
              None]
        for g in range(CG):
            b = g & 1
            if g + 1 < CG:
                cp[1 - b] = pltpu.async_copy(
                    xs_hbm.at[sidx.at[g + 1]], rowsbuf.at[1 - b], sems[1 - b])
            cp[b].wait()
            pltpu.sync_copy(rowsbuf.at[b], acc.at[didx.at[g]], add=True)
        return 0

    lax.fori_loop(0, NCHUNK, chunk, 0)
    plsc.subcore_barrier()
    pltpu.sync_copy(acc.at[pl.ds(sid * TROWS, TROWS)],
                    out_hbm.at[cid, pl.ds(sid * TROWS, TROWS)])


@functools.cache
def _scatter_call():
    return pl.kernel(
        _scatter_body,
        out_type=jax.ShapeDtypeStruct((NC, NPAD, D), f32),
        mesh=_mesh(),
        compiler_params=pltpu.CompilerParams(needs_layout_passes=False),
        scratch_types=[
            pltpu.SemaphoreType.DMA((2,)),
            pltpu.VMEM((CG, G), jnp.int32),
            pltpu.VMEM((CG, G), jnp.int32),
            pltpu.VMEM((2, G, D), f32),
            pltpu.VMEM_SHARED((NPAD, D), f32),
        ],
    )


def _score_body(h_hbm, src_hbm, dst_hbm, out_hbm,
                rsem, csem, sidx, didx, rowsbuf, colsbuf, sv):
    cid, sid, w = _worker_ids()
    rsems = (rsem.at[0], rsem.at[1])
    csems = (csem.at[0], csem.at[1])
    lanes = lax.broadcasted_iota(jnp.int32, (16,), 0)

    def chunk(c, _):
        base = w * GPW + c * CG
        pltpu.sync_copy(src_hbm.at[pl.ds(base, CG)], sidx)
        pltpu.sync_copy(dst_hbm.at[pl.ds(base, CG)], didx)
        cpr = [pltpu.async_copy(h_hbm.at[sidx.at[0]], rowsbuf.at[0], rsems[0]),
               None]
        cpc = [pltpu.async_copy(h_hbm.at[didx.at[0]], colsbuf.at[0], csems[0]),
               None]
        for g in range(CG):
            b = g & 1
            if g + 1 < CG:
                cpr[1 - b] = pltpu.async_copy(
                    h_hbm.at[sidx.at[g + 1]], rowsbuf.at[1 - b], rsems[1 - b])
                cpc[1 - b] = pltpu.async_copy(
                    h_hbm.at[didx.at[g + 1]], colsbuf.at[1 - b], csems[1 - b])
            cpr[b].wait()
            cpc[b].wait()

            def sub(s, _):
                vec = jnp.zeros((16,), f32)
                for i in range(16):
                    e = s * 16 + i
                    acc = rowsbuf[b, e, pl.ds(0, 16)] * colsbuf[b, e, pl.ds(0, 16)]
                    for k in range(1, D // 16):
                        acc = acc + (rowsbuf[b, e, pl.ds(k * 16, 16)]
                                     * colsbuf[b, e, pl.ds(k * 16, 16)])
                    vec = vec + jnp.where(lanes == i, jnp.sum(acc), 0.0)
                sv[pl.ds(s * 16, 16)] = 1.0 / (1.0 + jnp.exp(-vec))
                return 0

            lax.fori_loop(0, G // 16, sub, 0)
            pltpu.sync_copy(sv, out_hbm.at[pl.ds((base + g) * G, G)])
        return 0

    lax.fori_loop(0, NCHUNK, chunk, 0)


@functools.cache
def _score_call():
    return pl.kernel(
        _score_body,
        out_type=jax.ShapeDtypeStruct((EPAD,), f32),
        mesh=_mesh(),
        compiler_params=pltpu.CompilerParams(needs_layout_passes=False),
        scratch_types=[
            pltpu.SemaphoreType.DMA((2,)),
            pltpu.SemaphoreType.DMA((2,)),
            pltpu.VMEM((CG, G), jnp.int32),
            pltpu.VMEM((CG, G), jnp.int32),
            pltpu.VMEM((2, G, D), f32),
            pltpu.VMEM((2, G, D), f32),
            pltpu.VMEM((G,), f32),
        ],
    )


def _dinv_from(da_ref, db_ref):
    deg = da_ref[0, :, 0:1] + db_ref[0, :, 0:1] + 1.0
    return lax.rsqrt(deg)


def _tc1_body(x_ref, w_ref, da_ref, db_ref, o_ref):
    dinv = _dinv_from(da_ref, db_ref)
    o_ref[...] = jnp.dot(x_ref[...], w_ref[...],
                         preferred_element_type=f32) * dinv


def _tc2_body(a0_ref, a1_ref, xs_ref, b_ref, w_ref, da_ref, db_ref, o_ref):
    dinv = _dinv_from(da_ref, db_ref)
    h = dinv * (a0_ref[0] + a1_ref[0] + xs_ref[...]) + b_ref[...]
    h = jnp.maximum(h, 0.0)
    o_ref[...] = jnp.dot(h, w_ref[...], preferred_element_type=f32) * dinv


def _tc3_body(a0_ref, a1_ref, xs_ref, b_ref, da_ref, db_ref, o_ref):
    dinv = _dinv_from(da_ref, db_ref)
    o_ref[...] = dinv * (a0_ref[0] + a1_ref[0] + xs_ref[...]) + b_ref[...]


_blk = pl.BlockSpec((TBLK, D), lambda i: (i, 0))
_blkW = pl.BlockSpec((D, D), lambda i: (0, 0))
_blkb = pl.BlockSpec((1, D), lambda i: (0, 0))
_blk_dA = pl.BlockSpec((1, TBLK, 16), lambda i: (0, i, 0))
_blk_dB = pl.BlockSpec((1, TBLK, 16), lambda i: (1, i, 0))
_blk_aA = pl.BlockSpec((1, TBLK, D), lambda i: (0, i, 0))
_blk_aB = pl.BlockSpec((1, TBLK, D), lambda i: (1, i, 0))
_out_nd = jax.ShapeDtypeStruct((NPAD, D), f32)
_grid = (NPAD // TBLK,)


def _tc1(x_pad, W1, deg):
    return pl.pallas_call(
        _tc1_body, grid=_grid,
        in_specs=[_blk, _blkW, _blk_dA, _blk_dB],
        out_specs=_blk, out_shape=_out_nd,
    )(x_pad, W1, deg, deg)


def _tc2(acc1, xs1, b1, W2, deg):
    return pl.pallas_call(
        _tc2_body, grid=_grid,
        in_specs=[_blk_aA, _blk_aB, _blk, _blkb, _blkW, _blk_dA, _blk_dB],
        out_specs=_blk, out_shape=_out_nd,
    )(acc1, acc1, xs1, b1, W2, deg, deg)


def _tc3(acc2, xs2, b2, deg):
    return pl.pallas_call(
        _tc3_body, grid=_grid,
        in_specs=[_blk_aA, _blk_aB, _blk, _blkb, _blk_dA, _blk_dB],
        out_specs=_blk, out_shape=_out_nd,
    )(acc2, acc2, xs2, b2, deg, deg)


def kernel(x, edge_index, W1, b1, W2, b2):
    src = edge_index[0].astype(jnp.int32)
    dst = edge_index[1].astype(jnp.int32)
    pad = jnp.full((EPAD - E,), N, jnp.int32)
    src2d = jnp.concatenate([src, pad]).reshape(R2D, G)
    dst2d = jnp.concatenate([dst, pad]).reshape(R2D, G)
    x_pad = jnp.concatenate([x, jnp.zeros((NPAD - N, D), f32)], axis=0)

    deg = _deg_call()(dst2d)
    xs1 = _tc1(x_pad, W1, deg)
    acc1 = _scatter_call()(xs1, src2d, dst2d)
    xs2 = _tc2(acc1, xs1, b1.reshape(1, D), W2, deg)
    acc2 = _scatter_call()(xs2, src2d, dst2d)
    h2 = _tc3(acc2, xs2, b2.reshape(1, D), deg)
    scores = _score_call()(h2, src2d, dst2d)
    return scores[:E]


# speedup vs baseline: 9.4601x; 1.6972x over previous
import functools

import jax
import jax.numpy as jnp
from jax import lax
from jax.experimental import pallas as pl
from jax.experimental.pallas import tpu as pltpu
from jax.experimental.pallas import tpu_sc as plsc

f32 = jnp.float32

N = 10000
D = 128
E = 320000

NC = 2
NS = 16
NW = NC * NS

G = 128
GPW = 80
EPAD = NW * GPW * G
R2D = EPAD // G

NPAD = 10240
ZROWS = NPAD // NW
TROWS = NPAD // NS

TBLK = 2560

CG = 8
NCHUNK = GPW // CG

GA = 128
GB = 32

@functools.cache
def _mesh():
    return plsc.VectorSubcoreMesh(core_axis_name="c", subcore_axis_name="s",
                                  num_cores=NC, num_subcores=NS)


def _worker_ids():
    cid = lax.axis_index("c")
    sid = lax.axis_index("s")
    return cid, sid, cid * NS + sid


def _deg_body(dst_hbm, out_hbm, idx_v, ones_v, acc):
    cid, sid, w = _worker_ids()
    one = jnp.ones((16,), f32)
    zero = jnp.zeros((16,), f32)

    def fill(val):
        def go(i, _):
            ones_v[i] = val
            return 0
        lax.fori_loop(0, G, go, 0)

    fill(zero)
    for t in range(TROWS // G):
        pltpu.sync_copy(ones_v, acc.at[pl.ds(sid * TROWS + t * G, G)])
    fill(one)
    pltpu.sync_copy(dst_hbm.at[pl.ds(w * GPW, GPW)], idx_v)
    plsc.subcore_barrier()

    def body(j, _):
        pltpu.sync_copy(ones_v, acc.at[idx_v.at[j]], add=True)
        return 0

    lax.fori_loop(0, GPW, body, 0)
    plsc.subcore_barrier()
    pltpu.sync_copy(acc.at[pl.ds(sid * TROWS, TROWS)],
                    out_hbm.at[cid, pl.ds(sid * TROWS, TROWS)])


@functools.cache
def _deg_call():
    return pl.kernel(
        _deg_body,
        out_type=jax.ShapeDtypeStruct((NC, NPAD, 16), f32),
        mesh=_mesh(),
        compiler_params=pltpu.CompilerParams(needs_layout_passes=False),
        scratch_types=[
            pltpu.VMEM((GPW, G), jnp.int32),
            pltpu.VMEM((G, 16), f32),
            pltpu.VMEM_SHARED((NPAD, 16), f32),
        ],
    )


def _scatter_body(xs_hbm, src_hbm, dst_hbm, out_hbm,
                  sem, sidx, didx, rowsbuf, acc):
    cid, sid, w = _worker_ids()
    zero = jnp.zeros((16,), f32)
    sems = (sem.at[0], sem.at[1])

    def fill_z(i, _):
        for k in range(D // 16):
            rowsbuf[0, i, pl.ds(k * 16, 16)] = zero
        return 0

    lax.fori_loop(0, G, fill_z, 0)
    for t in range(TROWS // G):
        pltpu.sync_copy(rowsbuf.at[0], acc.at[pl.ds(sid * TROWS + t * G, G)])
    plsc.subcore_barrier()

    gbase = jnp.where(cid == 0, sid * GA, NS * GA + sid * GB)
    nchunks = jnp.where(cid == 0, GA // CG, GB // CG)

    def chunk(c, _):
        base = gbase + c * CG
        pltpu.sync_copy(src_hbm.at[pl.ds(base, CG)], sidx)
        pltpu.sync_copy(dst_hbm.at[pl.ds(base, CG)], didx)
        cp = [pltpu.async_copy(xs_hbm.at[sidx.at[0]], rowsbuf.at[0], sems[0]),
              None]
        for g in range(CG):
            b = g & 1
            if g + 1 < CG:
                cp[1 - b] = pltpu.async_copy(
                    xs_hbm.at[sidx.at[g + 1]], rowsbuf.at[1 - b], sems[1 - b])
            cp[b].wait()
            pltpu.sync_copy(rowsbuf.at[b], acc.at[didx.at[g]], add=True)
        return 0

    lax.fori_loop(0, nchunks, chunk, 0)
    plsc.subcore_barrier()
    pltpu.sync_copy(acc.at[pl.ds(sid * TROWS, TROWS)],
                    out_hbm.at[cid, pl.ds(sid * TROWS, TROWS)])


@functools.cache
def _scatter_call():
    return pl.kernel(
        _scatter_body,
        out_type=jax.ShapeDtypeStruct((NC, NPAD, D), f32),
        mesh=_mesh(),
        compiler_params=pltpu.CompilerParams(needs_layout_passes=False),
        scratch_types=[
            pltpu.SemaphoreType.DMA((2,)),
            pltpu.VMEM((CG, G), jnp.int32),
            pltpu.VMEM((CG, G), jnp.int32),
            pltpu.VMEM((2, G, D), f32),
            pltpu.VMEM_SHARED((NPAD, D), f32),
        ],
    )


def _score_body(h_hbm, src_hbm, dst_hbm, out_hbm,
                sem, sidx, didx, rows, cols, sv, hsh):
    cid, sid, w = _worker_ids()
    lanes = lax.broadcasted_iota(jnp.int32, (16,), 0)

    pltpu.sync_copy(h_hbm.at[pl.ds(sid * TROWS, TROWS)],
                    hsh.at[pl.ds(sid * TROWS, TROWS)])
    plsc.subcore_barrier()

    def chunk(c, _):
        base = w * GPW + c * CG
        pltpu.sync_copy(src_hbm.at[pl.ds(base, CG)], sidx)
        pltpu.sync_copy(dst_hbm.at[pl.ds(base, CG)], didx)
        for g in range(CG):
            c1 = pltpu.async_copy(hsh.at[sidx.at[g]], rows, sem.at[0])
            c2 = pltpu.async_copy(hsh.at[didx.at[g]], cols, sem.at[1])
            c1.wait()
            c2.wait()

            def sub(s, _):
                def edot(t, vec):
                    e = s * 16 + t
                    acc = rows[e, pl.ds(0, 16)] * cols[e, pl.ds(0, 16)]
                    for k in range(1, D // 16):
                        acc = acc + (rows[e, pl.ds(k * 16, 16)]
                                     * cols[e, pl.ds(k * 16, 16)])
                    sc = jnp.sum(acc)
                    return jnp.where(lanes == jnp.full((16,), t, jnp.int32),
                                     sc, vec)

                vec = lax.fori_loop(0, 16, edot, jnp.zeros((16,), f32))
                sv[pl.ds(s * 16, 16)] = 1.0 / (1.0 + jnp.exp(-vec))
                return 0

            lax.fori_loop(0, G // 16, sub, 0)
            pltpu.sync_copy(sv, out_hbm.at[pl.ds((base + g) * G, G)])
        return 0

    lax.fori_loop(0, NCHUNK, chunk, 0)


@functools.cache
def _score_call():
    return pl.kernel(
        _score_body,
        out_type=jax.ShapeDtypeStruct((EPAD,), f32),
        mesh=_mesh(),
        compiler_params=pltpu.CompilerParams(needs_layout_passes=False),
        scratch_types=[
            pltpu.SemaphoreType.DMA((2,)),
            pltpu.VMEM((CG, G), jnp.int32),
            pltpu.VMEM((CG, G), jnp.int32),
            pltpu.VMEM((G, D), f32),
            pltpu.VMEM((G, D), f32),
            pltpu.VMEM((G,), f32),
            pltpu.VMEM_SHARED((NPAD, D), f32),
        ],
    )


def _dinv_from(da_ref, db_ref):
    deg = da_ref[0, :, 0:1] + db_ref[0, :, 0:1] + 1.0
    return lax.rsqrt(deg)


def _tc1_body(x_ref, w_ref, da_ref, db_ref, o_ref):
    dinv = _dinv_from(da_ref, db_ref)
    o_ref[...] = jnp.dot(x_ref[...], w_ref[...],
                         preferred_element_type=f32) * dinv


def _tc2_body(a0_ref, a1_ref, xs_ref, b_ref, w_ref, da_ref, db_ref, o_ref):
    dinv = _dinv_from(da_ref, db_ref)
    h = dinv * (a0_ref[0] + a1_ref[0] + xs_ref[...]) + b_ref[...]
    h = jnp.maximum(h, 0.0)
    o_ref[...] = jnp.dot(h, w_ref[...], preferred_element_type=f32) * dinv


def _tc3_body(a0_ref, a1_ref, xs_ref, b_ref, da_ref, db_ref, o_ref):
    dinv = _dinv_from(da_ref, db_ref)
    o_ref[...] = dinv * (a0_ref[0] + a1_ref[0] + xs_ref[...]) + b_ref[...]


_blk = pl.BlockSpec((TBLK, D), lambda i: (i, 0))
_blkW = pl.BlockSpec((D, D), lambda i: (0, 0))
_blkb = pl.BlockSpec((1, D), lambda i: (0, 0))
_blk_dA = pl.BlockSpec((1, TBLK, 16), lambda i: (0, i, 0))
_blk_dB = pl.BlockSpec((1, TBLK, 16), lambda i: (1, i, 0))
_blk_aA = pl.BlockSpec((1, TBLK, D), lambda i: (0, i, 0))
_blk_aB = pl.BlockSpec((1, TBLK, D), lambda i: (1, i, 0))
_out_nd = jax.ShapeDtypeStruct((NPAD, D), f32)
_grid = (NPAD // TBLK,)


def _tc1(x_pad, W1, deg):
    return pl.pallas_call(
        _tc1_body, grid=_grid,
        in_specs=[_blk, _blkW, _blk_dA, _blk_dB],
        out_specs=_blk, out_shape=_out_nd,
    )(x_pad, W1, deg, deg)


def _tc2(acc1, xs1, b1, W2, deg):
    return pl.pallas_call(
        _tc2_body, grid=_grid,
        in_specs=[_blk_aA, _blk_aB, _blk, _blkb, _blkW, _blk_dA, _blk_dB],
        out_specs=_blk, out_shape=_out_nd,
    )(acc1, acc1, xs1, b1, W2, deg, deg)


def _tc3(acc2, xs2, b2, deg):
    return pl.pallas_call(
        _tc3_body, grid=_grid,
        in_specs=[_blk_aA, _blk_aB, _blk, _blkb, _blk_dA, _blk_dB],
        out_specs=_blk, out_shape=_out_nd,
    )(acc2, acc2, xs2, b2, deg, deg)


def kernel(x, edge_index, W1, b1, W2, b2):
    src = edge_index[0].astype(jnp.int32)
    dst = edge_index[1].astype(jnp.int32)
    pad = jnp.full((EPAD - E,), N, jnp.int32)
    src2d = jnp.concatenate([src, pad]).reshape(R2D, G)
    dst2d = jnp.concatenate([dst, pad]).reshape(R2D, G)
    x_pad = jnp.concatenate([x, jnp.zeros((NPAD - N, D), f32)], axis=0)

    deg = _deg_call()(dst2d)
    xs1 = _tc1(x_pad, W1, deg)
    acc1 = _scatter_call()(xs1, src2d, dst2d)
    xs2 = _tc2(acc1, xs1, b1.reshape(1, D), W2, deg)
    acc2 = _scatter_call()(xs2, src2d, dst2d)
    h2 = _tc3(acc2, xs2, b2.reshape(1, D), deg)
    scores = _score_call()(h2, src2d, dst2d)
    return scores[:E]
